# Initial kernel scaffold; baseline (speedup 1.0000x reference)
import jax
import jax.numpy as jnp
from jax.experimental import pallas as pl


def kernel(uids, pos, neg, E_u_0, E_i_0, adj_row, adj_col, adj_val):
    raise NotImplementedError("write your pallas kernel here")


# speedup vs baseline: 3.1140x; 3.1140x over previous
import functools
import jax
import jax.numpy as jnp
from jax import lax
from jax.experimental import pallas as pl
from jax.experimental.pallas import tpu as pltpu
from jax.experimental.pallas import tpu_sc as plsc

N_U = 100000
N_I = 100000
D = 64
NQ = 4
QL = 16
NNZ = 1000000
NNZ_P = 1 << 20
EW = 1024
ROWS_PER_WIN = EW // 128
N_WIN = NNZ_P // EW
WIN_PER_TILE = N_WIN // 16
BATCH = 4096
LAMBDA_1 = 1e-4

_mesh = plsc.VectorSubcoreMesh(core_axis_name="c", subcore_axis_name="s")
_sc_params = pltpu.CompilerParams(use_tc_tiling_on_sc=False)


def _spmm_body(dst_hbm, src_hbm, val_hbm, tab_hbm, out_hbm,
               acc, dstv, srcv, valv, rows, zbuf):
    c = lax.axis_index("c")
    s = lax.axis_index("s")

    def _z(i, _):
        zbuf[i, :] = jnp.zeros((QL,), jnp.float32)
        return 0
    lax.fori_loop(0, zbuf.shape[0], _z, 0)

    zrows = zbuf.shape[0]
    STRIPE = 6248
    n_zcopy = STRIPE // zrows

    for p in range(2):
        for k in range(n_zcopy):
            pltpu.sync_copy(zbuf, acc.at[pl.ds(s * STRIPE + k * zrows,
                                               zrows)])

        @pl.when(s == 0)
        def _():
            pltpu.sync_copy(zbuf.at[pl.ds(0, 32)],
                            acc.at[pl.ds(16 * STRIPE, 32)])
        plsc.subcore_barrier()

        for cc in range(2):
            q = 2 * p + cc

            @pl.when(c == cc)
            def _():
                def win(w, _):
                    g = w * 16 + s
                    r0 = g * ROWS_PER_WIN
                    pltpu.sync_copy(dst_hbm.at[pl.ds(r0, ROWS_PER_WIN)], dstv)
                    pltpu.sync_copy(src_hbm.at[pl.ds(r0, ROWS_PER_WIN)], srcv)
                    pltpu.sync_copy(val_hbm.at[pl.ds(r0, ROWS_PER_WIN)], valv)
                    for j in range(ROWS_PER_WIN):
                        pltpu.sync_copy(tab_hbm.at[q].at[srcv.at[j]],
                                        rows.at[pl.ds(j * 128, 128)])
                    for j in range(ROWS_PER_WIN):
                        def scale(m, _):
                            vv = valv[j, pl.ds(m * 16, 16)]
                            for k in range(16):
                                e = j * 128 + m * 16 + k
                                rows[e, :] = vv[k] * rows[e, :]
                            return 0
                        lax.fori_loop(0, 8, scale, 0)
                    for j in range(ROWS_PER_WIN):
                        pltpu.sync_copy(rows.at[pl.ds(j * 128, 128)],
                                        acc.at[dstv.at[j]], add=True)
                    return 0
                lax.fori_loop(0, WIN_PER_TILE, win, 0)

        plsc.subcore_barrier()

        for cc in range(2):
            q = 2 * p + cc

            @pl.when(c == cc)
            def _():
                pltpu.sync_copy(
                    acc.at[pl.ds(s * STRIPE, STRIPE)],
                    out_hbm.at[q].at[pl.ds(s * STRIPE, STRIPE)])

                @pl.when(s == 0)
                def _():
                    pltpu.sync_copy(
                        acc.at[pl.ds(16 * STRIPE, 32)],
                        out_hbm.at[q].at[pl.ds(16 * STRIPE, 32)])

        plsc.subcore_barrier()


@functools.partial(
    pl.kernel,
    out_type=jax.ShapeDtypeStruct((NQ, N_U, QL), jnp.float32),
    mesh=_mesh,
    compiler_params=_sc_params,
    scratch_types=[
        pltpu.VMEM_SHARED((N_U, QL), jnp.float32),
        pltpu.VMEM((ROWS_PER_WIN, 128), jnp.int32),
        pltpu.VMEM((ROWS_PER_WIN, 128), jnp.int32),
        pltpu.VMEM((ROWS_PER_WIN, 128), jnp.float32),
        pltpu.VMEM((EW, QL), jnp.float32),
        pltpu.VMEM((142, QL), jnp.float32),
    ],
)
def _spmm(dst_hbm, src_hbm, val_hbm, tab_hbm, out_hbm,
          acc, dstv, srcv, valv, rows, zbuf):
    _spmm_body(dst_hbm, src_hbm, val_hbm, tab_hbm, out_hbm,
               acc, dstv, srcv, valv, rows, zbuf)


@functools.partial(
    pl.kernel,
    out_type=[jax.ShapeDtypeStruct((32, 128, QL), jnp.float32),
              jax.ShapeDtypeStruct((32, 128, QL), jnp.float32),
              jax.ShapeDtypeStruct((32, 128, QL), jnp.float32)],
    mesh=_mesh,
    compiler_params=_sc_params,
    scratch_types=(
        [pltpu.VMEM((1, 128), jnp.int32) for _ in range(3)]
        + [pltpu.VMEM((NQ, 128, QL), jnp.float32) for _ in range(9)]
        + [pltpu.VMEM((128, QL), jnp.float32) for _ in range(3)]
    ),
)
def _batch(uid_hbm, pos_hbm, neg_hbm,
           eu0, zu1, zu2, ei0, zi1, zi2,
           ps_hbm, ns_hbm, rg_hbm,
           iu, ip, inn,
           bu0, bu1, bu2, bp0, bp1, bp2, bn0, bn1, bn2,
           ops, ons, org):
    c = lax.axis_index("c")
    s = lax.axis_index("s")
    wid = 2 * s + c

    pltpu.sync_copy(uid_hbm.at[wid], iu)
    pltpu.sync_copy(pos_hbm.at[wid], ip)
    pltpu.sync_copy(neg_hbm.at[wid], inn)

    for q in range(NQ):
        pltpu.sync_copy(eu0.at[q].at[iu.at[0]], bu0.at[q])
        pltpu.sync_copy(zu1.at[q].at[iu.at[0]], bu1.at[q])
        pltpu.sync_copy(zu2.at[q].at[iu.at[0]], bu2.at[q])
        pltpu.sync_copy(ei0.at[q].at[ip.at[0]], bp0.at[q])
        pltpu.sync_copy(zi1.at[q].at[ip.at[0]], bp1.at[q])
        pltpu.sync_copy(zi2.at[q].at[ip.at[0]], bp2.at[q])
        pltpu.sync_copy(ei0.at[q].at[inn.at[0]], bn0.at[q])
        pltpu.sync_copy(zi1.at[q].at[inn.at[0]], bn1.at[q])
        pltpu.sync_copy(zi2.at[q].at[inn.at[0]], bn2.at[q])

    def sample(b, _):
        accp = jnp.zeros((QL,), jnp.float32)
        accn = jnp.zeros((QL,), jnp.float32)
        accr = jnp.zeros((QL,), jnp.float32)
        for q in range(NQ):
            u0 = bu0[q, b, :]
            p0 = bp0[q, b, :]
            n0 = bn0[q, b, :]
            u = u0 + bu1[q, b, :] + bu2[q, b, :]
            pv = p0 + bp1[q, b, :] + bp2[q, b, :]
            nv = n0 + bn1[q, b, :] + bn2[q, b, :]
            accp = accp + u * pv
            accn = accn + u * nv
            accr = accr + u0 * u0 + p0 * p0 + n0 * n0
        ops[b, :] = accp
        ons[b, :] = accn
        org[b, :] = accr
        return 0
    lax.fori_loop(0, 128, sample, 0)

    pltpu.sync_copy(ops, ps_hbm.at[wid])
    pltpu.sync_copy(ons, ns_hbm.at[wid])
    pltpu.sync_copy(org, rg_hbm.at[wid])


def _finalize_body(ps_ref, ns_ref, rg_ref, loss_ref, lossr_ref):
    d = jnp.sum(ps_ref[...] - ns_ref[...], axis=-1)
    loss_r = -jnp.mean(jnp.log(jax.nn.sigmoid(d)))
    loss_reg = 0.5 * jnp.sum(rg_ref[...]) / float(BATCH) * LAMBDA_1
    lossr_ref[0, 0] = loss_r
    loss_ref[0, 0] = loss_r + loss_reg


def kernel(uids, pos, neg, E_u_0, E_i_0, adj_row, adj_col, adj_val):
    padn = NNZ_P - NNZ
    pad_idx = (jnp.arange(padn, dtype=jnp.int32) * 61) % 99991
    rowp = jnp.concatenate([adj_row, pad_idx]).reshape(NNZ_P // 128, 128)
    colp = jnp.concatenate([adj_col, pad_idx]).reshape(NNZ_P // 128, 128)
    valp = jnp.concatenate([adj_val, jnp.zeros((padn,), jnp.float32)]
                           ).reshape(NNZ_P // 128, 128)

    eu0 = E_u_0.reshape(N_U, NQ, QL).transpose(1, 0, 2)
    ei0 = E_i_0.reshape(N_I, NQ, QL).transpose(1, 0, 2)

    zu1 = _spmm(rowp, colp, valp, ei0)
    zi1 = _spmm(colp, rowp, valp, eu0)
    zu2 = _spmm(rowp, colp, valp, zi1)
    zi2 = _spmm(colp, rowp, valp, zu1)

    u32 = uids.reshape(32, 1, 128)
    p32 = pos.reshape(32, 1, 128)
    n32 = neg.reshape(32, 1, 128)
    ps, ns, rg = _batch(u32, p32, n32, eu0, zu1, zu2, ei0, zi1, zi2)

    loss, loss_r = pl.pallas_call(
        _finalize_body,
        out_shape=[jax.ShapeDtypeStruct((1, 1), jnp.float32),
                   jax.ShapeDtypeStruct((1, 1), jnp.float32)],
        out_specs=[pl.BlockSpec(memory_space=pltpu.SMEM),
                   pl.BlockSpec(memory_space=pltpu.SMEM)],
    )(ps.reshape(BATCH, QL), ns.reshape(BATCH, QL), rg.reshape(BATCH, QL))
    return (loss[0, 0], loss_r[0, 0])


# speedup vs baseline: 6.7434x; 2.1655x over previous
import functools
import jax
import jax.numpy as jnp
from jax import lax
from jax.experimental import pallas as pl
from jax.experimental.pallas import tpu as pltpu
from jax.experimental.pallas import tpu_sc as plsc

N_U = 100000
N_I = 100000
D = 64
NQ = 4
QL = 16
NNZ = 1000000
NNZ_P = 1 << 20
EW = 1024
ROWS_PER_WIN = EW // 128
N_WIN = NNZ_P // EW
WIN_PER_TILE = N_WIN // 16
BATCH = 4096
LAMBDA_1 = 1e-4

_mesh = plsc.VectorSubcoreMesh(core_axis_name="c", subcore_axis_name="s")
_sc_params = pltpu.CompilerParams(use_tc_tiling_on_sc=False)


def _spmm_body(dst_hbm, src_hbm, val_hbm, tab_hbm, out_hbm,
               acc, dstv, srcv, valv, rows, zbuf, lsem, gsem, ssem):
    c = lax.axis_index("c")
    s = lax.axis_index("s")

    def _z(i, _):
        zbuf[i, :] = jnp.zeros((QL,), jnp.float32)
        return 0
    lax.fori_loop(0, zbuf.shape[0], _z, 0)

    zrows = zbuf.shape[0]
    STRIPE = 6248
    n_zcopy = STRIPE // zrows

    for p in range(2):
        for k in range(n_zcopy):
            pltpu.sync_copy(zbuf, acc.at[pl.ds(s * STRIPE + k * zrows,
                                               zrows)])

        @pl.when(s == 0)
        def _():
            pltpu.sync_copy(zbuf.at[pl.ds(0, 32)],
                            acc.at[pl.ds(16 * STRIPE, 32)])
        plsc.subcore_barrier()

        for cc in range(2):
            q = 2 * p + cc

            @pl.when(c == cc)
            def _():
                def win(w, _):
                    g = w * 16 + s
                    r0 = g * ROWS_PER_WIN
                    cps = [
                        pltpu.async_copy(dst_hbm.at[pl.ds(r0, ROWS_PER_WIN)],
                                         dstv, lsem),
                        pltpu.async_copy(src_hbm.at[pl.ds(r0, ROWS_PER_WIN)],
                                         srcv, lsem),
                        pltpu.async_copy(val_hbm.at[pl.ds(r0, ROWS_PER_WIN)],
                                         valv, lsem),
                    ]
                    for cp in cps:
                        cp.wait()
                    gps = [
                        pltpu.async_copy(tab_hbm.at[q].at[srcv.at[j]],
                                         rows.at[pl.ds(j * 128, 128)], gsem)
                        for j in range(ROWS_PER_WIN)
                    ]
                    for cp in gps:
                        cp.wait()
                    for j in range(ROWS_PER_WIN):
                        def scale(m, _):
                            vv = valv[j, pl.ds(m * 16, 16)]
                            for k in range(16):
                                e = j * 128 + m * 16 + k
                                rows[e, :] = vv[k] * rows[e, :]
                            return 0
                        lax.fori_loop(0, 8, scale, 0)
                    sps = [
                        pltpu.async_copy(rows.at[pl.ds(j * 128, 128)],
                                         acc.at[dstv.at[j]], ssem, add=True)
                        for j in range(ROWS_PER_WIN)
                    ]
                    for cp in sps:
                        cp.wait()
                    return 0
                lax.fori_loop(0, WIN_PER_TILE, win, 0)

        plsc.subcore_barrier()

        for cc in range(2):
            q = 2 * p + cc

            @pl.when(c == cc)
            def _():
                pltpu.sync_copy(
                    acc.at[pl.ds(s * STRIPE, STRIPE)],
                    out_hbm.at[q].at[pl.ds(s * STRIPE, STRIPE)])

                @pl.when(s == 0)
                def _():
                    pltpu.sync_copy(
                        acc.at[pl.ds(16 * STRIPE, 32)],
                        out_hbm.at[q].at[pl.ds(16 * STRIPE, 32)])

        plsc.subcore_barrier()


@functools.partial(
    pl.kernel,
    out_type=jax.ShapeDtypeStruct((NQ, N_U, QL), jnp.float32),
    mesh=_mesh,
    compiler_params=_sc_params,
    scratch_types=[
        pltpu.VMEM_SHARED((N_U, QL), jnp.float32),
        pltpu.VMEM((ROWS_PER_WIN, 128), jnp.int32),
        pltpu.VMEM((ROWS_PER_WIN, 128), jnp.int32),
        pltpu.VMEM((ROWS_PER_WIN, 128), jnp.float32),
        pltpu.VMEM((EW, QL), jnp.float32),
        pltpu.VMEM((142, QL), jnp.float32),
        pltpu.SemaphoreType.DMA,
        pltpu.SemaphoreType.DMA,
        pltpu.SemaphoreType.DMA,
    ],
)
def _spmm(dst_hbm, src_hbm, val_hbm, tab_hbm, out_hbm,
          acc, dstv, srcv, valv, rows, zbuf, lsem, gsem, ssem):
    _spmm_body(dst_hbm, src_hbm, val_hbm, tab_hbm, out_hbm,
               acc, dstv, srcv, valv, rows, zbuf, lsem, gsem, ssem)


@functools.partial(
    pl.kernel,
    out_type=[jax.ShapeDtypeStruct((32, 128, QL), jnp.float32),
              jax.ShapeDtypeStruct((32, 128, QL), jnp.float32),
              jax.ShapeDtypeStruct((32, 128, QL), jnp.float32)],
    mesh=_mesh,
    compiler_params=_sc_params,
    scratch_types=(
        [pltpu.VMEM((1, 128), jnp.int32) for _ in range(3)]
        + [pltpu.VMEM((NQ, 128, QL), jnp.float32) for _ in range(9)]
        + [pltpu.VMEM((128, QL), jnp.float32) for _ in range(3)]
    ),
)
def _batch(uid_hbm, pos_hbm, neg_hbm,
           eu0, zu1, zu2, ei0, zi1, zi2,
           ps_hbm, ns_hbm, rg_hbm,
           iu, ip, inn,
           bu0, bu1, bu2, bp0, bp1, bp2, bn0, bn1, bn2,
           ops, ons, org):
    c = lax.axis_index("c")
    s = lax.axis_index("s")
    wid = 2 * s + c

    pltpu.sync_copy(uid_hbm.at[wid], iu)
    pltpu.sync_copy(pos_hbm.at[wid], ip)
    pltpu.sync_copy(neg_hbm.at[wid], inn)

    for q in range(NQ):
        pltpu.sync_copy(eu0.at[q].at[iu.at[0]], bu0.at[q])
        pltpu.sync_copy(zu1.at[q].at[iu.at[0]], bu1.at[q])
        pltpu.sync_copy(zu2.at[q].at[iu.at[0]], bu2.at[q])
        pltpu.sync_copy(ei0.at[q].at[ip.at[0]], bp0.at[q])
        pltpu.sync_copy(zi1.at[q].at[ip.at[0]], bp1.at[q])
        pltpu.sync_copy(zi2.at[q].at[ip.at[0]], bp2.at[q])
        pltpu.sync_copy(ei0.at[q].at[inn.at[0]], bn0.at[q])
        pltpu.sync_copy(zi1.at[q].at[inn.at[0]], bn1.at[q])
        pltpu.sync_copy(zi2.at[q].at[inn.at[0]], bn2.at[q])

    def sample(b, _):
        accp = jnp.zeros((QL,), jnp.float32)
        accn = jnp.zeros((QL,), jnp.float32)
        accr = jnp.zeros((QL,), jnp.float32)
        for q in range(NQ):
            u0 = bu0[q, b, :]
            p0 = bp0[q, b, :]
            n0 = bn0[q, b, :]
            u = u0 + bu1[q, b, :] + bu2[q, b, :]
            pv = p0 + bp1[q, b, :] + bp2[q, b, :]
            nv = n0 + bn1[q, b, :] + bn2[q, b, :]
            accp = accp + u * pv
            accn = accn + u * nv
            accr = accr + u0 * u0 + p0 * p0 + n0 * n0
        ops[b, :] = accp
        ons[b, :] = accn
        org[b, :] = accr
        return 0
    lax.fori_loop(0, 128, sample, 0)

    pltpu.sync_copy(ops, ps_hbm.at[wid])
    pltpu.sync_copy(ons, ns_hbm.at[wid])
    pltpu.sync_copy(org, rg_hbm.at[wid])


def _finalize_body(ps_ref, ns_ref, rg_ref, loss_ref, lossr_ref):
    d = jnp.sum(ps_ref[...] - ns_ref[...], axis=-1)
    loss_r = -jnp.mean(jnp.log(jax.nn.sigmoid(d)))
    loss_reg = 0.5 * jnp.sum(rg_ref[...]) / float(BATCH) * LAMBDA_1
    lossr_ref[0, 0] = loss_r
    loss_ref[0, 0] = loss_r + loss_reg


def kernel(uids, pos, neg, E_u_0, E_i_0, adj_row, adj_col, adj_val):
    padn = NNZ_P - NNZ
    pad_idx = (jnp.arange(padn, dtype=jnp.int32) * 61) % 99991
    rowp = jnp.concatenate([adj_row, pad_idx]).reshape(NNZ_P // 128, 128)
    colp = jnp.concatenate([adj_col, pad_idx]).reshape(NNZ_P // 128, 128)
    valp = jnp.concatenate([adj_val, jnp.zeros((padn,), jnp.float32)]
                           ).reshape(NNZ_P // 128, 128)

    eu0 = E_u_0.reshape(N_U, NQ, QL).transpose(1, 0, 2)
    ei0 = E_i_0.reshape(N_I, NQ, QL).transpose(1, 0, 2)

    zu1 = _spmm(rowp, colp, valp, ei0)
    zi1 = _spmm(colp, rowp, valp, eu0)
    zu2 = _spmm(rowp, colp, valp, zi1)
    zi2 = _spmm(colp, rowp, valp, zu1)

    u32 = uids.reshape(32, 1, 128)
    p32 = pos.reshape(32, 1, 128)
    n32 = neg.reshape(32, 1, 128)
    ps, ns, rg = _batch(u32, p32, n32, eu0, zu1, zu2, ei0, zi1, zi2)

    loss, loss_r = pl.pallas_call(
        _finalize_body,
        out_shape=[jax.ShapeDtypeStruct((1, 1), jnp.float32),
                   jax.ShapeDtypeStruct((1, 1), jnp.float32)],
        out_specs=[pl.BlockSpec(memory_space=pltpu.SMEM),
                   pl.BlockSpec(memory_space=pltpu.SMEM)],
    )(ps.reshape(BATCH, QL), ns.reshape(BATCH, QL), rg.reshape(BATCH, QL))
    return (loss[0, 0], loss_r[0, 0])


# speedup vs baseline: 6.7891x; 1.0068x over previous
import functools
import jax
import jax.numpy as jnp
from jax import lax
from jax.experimental import pallas as pl
from jax.experimental.pallas import tpu as pltpu
from jax.experimental.pallas import tpu_sc as plsc

N_U = 100000
N_I = 100000
D = 64
NQ = 4
QL = 16
NNZ = 1000000
NNZ_P = 1 << 20
EW = 1024
ROWS_PER_WIN = EW // 128
N_WIN = NNZ_P // EW
WIN_PER_TILE = N_WIN // 16
BATCH = 4096
LAMBDA_1 = 1e-4

_mesh = plsc.VectorSubcoreMesh(core_axis_name="c", subcore_axis_name="s")
_sc_params = pltpu.CompilerParams(use_tc_tiling_on_sc=False)


def _spmm_body(dst_hbm, src_hbm, val_hbm, tab_hbm, out_hbm,
               acc, dstv, srcv, valv, rows, zbuf, lsem, gsem, ssem):
    c = lax.axis_index("c")
    s = lax.axis_index("s")

    def _z(i, _):
        zbuf[i, :] = jnp.zeros((QL,), jnp.float32)
        return 0
    lax.fori_loop(0, zbuf.shape[0], _z, 0)

    zrows = zbuf.shape[0]
    STRIPE = 6248
    n_zcopy = STRIPE // zrows

    for p in range(2):
        for k in range(n_zcopy):
            pltpu.sync_copy(zbuf, acc.at[pl.ds(s * STRIPE + k * zrows,
                                               zrows)])

        @pl.when(s == 0)
        def _():
            pltpu.sync_copy(zbuf.at[pl.ds(0, 32)],
                            acc.at[pl.ds(16 * STRIPE, 32)])
        plsc.subcore_barrier()

        for cc in range(2):
            q = 2 * p + cc

            @pl.when(c == cc)
            def _():
                def win(w, _):
                    g = w * 16 + s
                    e0 = g * EW
                    cps = [
                        pltpu.async_copy(dst_hbm.at[pl.ds(e0, EW)],
                                         dstv, lsem),
                        pltpu.async_copy(src_hbm.at[pl.ds(e0, EW)],
                                         srcv, lsem),
                        pltpu.async_copy(val_hbm.at[pl.ds(e0, EW)],
                                         valv, lsem),
                    ]
                    for cp in cps:
                        cp.wait()
                    pltpu.async_copy(tab_hbm.at[q].at[srcv], rows,
                                     gsem).wait()
                    def scale(m, _):
                        vv = valv[pl.ds(m * 16, 16)]
                        for k in range(16):
                            rows[m * 16 + k, :] = vv[k] * rows[m * 16 + k, :]
                        return 0
                    lax.fori_loop(0, EW // 16, scale, 0)
                    pltpu.async_copy(rows, acc.at[dstv], ssem,
                                     add=True).wait()
                    return 0
                lax.fori_loop(0, WIN_PER_TILE, win, 0)

        plsc.subcore_barrier()

        for cc in range(2):
            q = 2 * p + cc

            @pl.when(c == cc)
            def _():
                pltpu.sync_copy(
                    acc.at[pl.ds(s * STRIPE, STRIPE)],
                    out_hbm.at[q].at[pl.ds(s * STRIPE, STRIPE)])

                @pl.when(s == 0)
                def _():
                    pltpu.sync_copy(
                        acc.at[pl.ds(16 * STRIPE, 32)],
                        out_hbm.at[q].at[pl.ds(16 * STRIPE, 32)])

        plsc.subcore_barrier()


@functools.partial(
    pl.kernel,
    out_type=jax.ShapeDtypeStruct((NQ, N_U, QL), jnp.float32),
    mesh=_mesh,
    compiler_params=_sc_params,
    scratch_types=[
        pltpu.VMEM_SHARED((N_U, QL), jnp.float32),
        pltpu.VMEM((EW,), jnp.int32),
        pltpu.VMEM((EW,), jnp.int32),
        pltpu.VMEM((EW,), jnp.float32),
        pltpu.VMEM((EW, QL), jnp.float32),
        pltpu.VMEM((142, QL), jnp.float32),
        pltpu.SemaphoreType.DMA,
        pltpu.SemaphoreType.DMA,
        pltpu.SemaphoreType.DMA,
    ],
)
def _spmm(dst_hbm, src_hbm, val_hbm, tab_hbm, out_hbm,
          acc, dstv, srcv, valv, rows, zbuf, lsem, gsem, ssem):
    _spmm_body(dst_hbm, src_hbm, val_hbm, tab_hbm, out_hbm,
               acc, dstv, srcv, valv, rows, zbuf, lsem, gsem, ssem)


@functools.partial(
    pl.kernel,
    out_type=[jax.ShapeDtypeStruct((32, 128, QL), jnp.float32),
              jax.ShapeDtypeStruct((32, 128, QL), jnp.float32),
              jax.ShapeDtypeStruct((32, 128, QL), jnp.float32)],
    mesh=_mesh,
    compiler_params=_sc_params,
    scratch_types=(
        [pltpu.VMEM((1, 128), jnp.int32) for _ in range(3)]
        + [pltpu.VMEM((NQ, 128, QL), jnp.float32) for _ in range(9)]
        + [pltpu.VMEM((128, QL), jnp.float32) for _ in range(3)]
    ),
)
def _batch(uid_hbm, pos_hbm, neg_hbm,
           eu0, zu1, zu2, ei0, zi1, zi2,
           ps_hbm, ns_hbm, rg_hbm,
           iu, ip, inn,
           bu0, bu1, bu2, bp0, bp1, bp2, bn0, bn1, bn2,
           ops, ons, org):
    c = lax.axis_index("c")
    s = lax.axis_index("s")
    wid = 2 * s + c

    pltpu.sync_copy(uid_hbm.at[wid], iu)
    pltpu.sync_copy(pos_hbm.at[wid], ip)
    pltpu.sync_copy(neg_hbm.at[wid], inn)

    for q in range(NQ):
        pltpu.sync_copy(eu0.at[q].at[iu.at[0]], bu0.at[q])
        pltpu.sync_copy(zu1.at[q].at[iu.at[0]], bu1.at[q])
        pltpu.sync_copy(zu2.at[q].at[iu.at[0]], bu2.at[q])
        pltpu.sync_copy(ei0.at[q].at[ip.at[0]], bp0.at[q])
        pltpu.sync_copy(zi1.at[q].at[ip.at[0]], bp1.at[q])
        pltpu.sync_copy(zi2.at[q].at[ip.at[0]], bp2.at[q])
        pltpu.sync_copy(ei0.at[q].at[inn.at[0]], bn0.at[q])
        pltpu.sync_copy(zi1.at[q].at[inn.at[0]], bn1.at[q])
        pltpu.sync_copy(zi2.at[q].at[inn.at[0]], bn2.at[q])

    def sample(b, _):
        accp = jnp.zeros((QL,), jnp.float32)
        accn = jnp.zeros((QL,), jnp.float32)
        accr = jnp.zeros((QL,), jnp.float32)
        for q in range(NQ):
            u0 = bu0[q, b, :]
            p0 = bp0[q, b, :]
            n0 = bn0[q, b, :]
            u = u0 + bu1[q, b, :] + bu2[q, b, :]
            pv = p0 + bp1[q, b, :] + bp2[q, b, :]
            nv = n0 + bn1[q, b, :] + bn2[q, b, :]
            accp = accp + u * pv
            accn = accn + u * nv
            accr = accr + u0 * u0 + p0 * p0 + n0 * n0
        ops[b, :] = accp
        ons[b, :] = accn
        org[b, :] = accr
        return 0
    lax.fori_loop(0, 128, sample, 0)

    pltpu.sync_copy(ops, ps_hbm.at[wid])
    pltpu.sync_copy(ons, ns_hbm.at[wid])
    pltpu.sync_copy(org, rg_hbm.at[wid])


def _finalize_body(ps_ref, ns_ref, rg_ref, loss_ref, lossr_ref):
    d = jnp.sum(ps_ref[...] - ns_ref[...], axis=-1)
    loss_r = -jnp.mean(jnp.log(jax.nn.sigmoid(d)))
    loss_reg = 0.5 * jnp.sum(rg_ref[...]) / float(BATCH) * LAMBDA_1
    lossr_ref[0, 0] = loss_r
    loss_ref[0, 0] = loss_r + loss_reg


def kernel(uids, pos, neg, E_u_0, E_i_0, adj_row, adj_col, adj_val):
    padn = NNZ_P - NNZ
    pad_idx = (jnp.arange(padn, dtype=jnp.int32) * 61) % 99991
    rowp = jnp.concatenate([adj_row, pad_idx])
    colp = jnp.concatenate([adj_col, pad_idx])
    valp = jnp.concatenate([adj_val, jnp.zeros((padn,), jnp.float32)])

    eu0 = E_u_0.reshape(N_U, NQ, QL).transpose(1, 0, 2)
    ei0 = E_i_0.reshape(N_I, NQ, QL).transpose(1, 0, 2)

    zu1 = _spmm(rowp, colp, valp, ei0)
    zi1 = _spmm(colp, rowp, valp, eu0)
    zu2 = _spmm(rowp, colp, valp, zi1)
    zi2 = _spmm(colp, rowp, valp, zu1)

    u32 = uids.reshape(32, 1, 128)
    p32 = pos.reshape(32, 1, 128)
    n32 = neg.reshape(32, 1, 128)
    ps, ns, rg = _batch(u32, p32, n32, eu0, zu1, zu2, ei0, zi1, zi2)

    loss, loss_r = pl.pallas_call(
        _finalize_body,
        out_shape=[jax.ShapeDtypeStruct((1, 1), jnp.float32),
                   jax.ShapeDtypeStruct((1, 1), jnp.float32)],
        out_specs=[pl.BlockSpec(memory_space=pltpu.SMEM),
                   pl.BlockSpec(memory_space=pltpu.SMEM)],
    )(ps.reshape(BATCH, QL), ns.reshape(BATCH, QL), rg.reshape(BATCH, QL))
    return (loss[0, 0], loss_r[0, 0])


# speedup vs baseline: 8.1862x; 1.2058x over previous
import functools
import jax
import jax.numpy as jnp
from jax import lax
from jax.experimental import pallas as pl
from jax.experimental.pallas import tpu as pltpu
from jax.experimental.pallas import tpu_sc as plsc

N_U = 100000
N_I = 100000
D = 64
NQ = 4
QL = 16
NNZ = 1000000
NNZ_P = 1 << 20
EW = 1024
ROWS_PER_WIN = EW // 128
NCH = 4
CH = EW // NCH
N_WIN = NNZ_P // EW
WIN_PER_TILE = N_WIN // 16
BATCH = 4096
LAMBDA_1 = 1e-4

_mesh = plsc.VectorSubcoreMesh(core_axis_name="c", subcore_axis_name="s")
_sc_params = pltpu.CompilerParams(use_tc_tiling_on_sc=False)


def _spmm_body(dst_hbm, src_hbm, val_hbm, tab_hbm, out_hbm,
               acc, dstv, srcv, valv, rows, zbuf, lsem, gsem, ssem):
    c = lax.axis_index("c")
    s = lax.axis_index("s")

    def _z(i, _):
        zbuf[i, :] = jnp.zeros((QL,), jnp.float32)
        return 0
    lax.fori_loop(0, zbuf.shape[0], _z, 0)

    zrows = zbuf.shape[0]
    STRIPE = 6248
    n_zcopy = STRIPE // zrows

    for p in range(2):
        for k in range(n_zcopy):
            pltpu.sync_copy(zbuf, acc.at[pl.ds(s * STRIPE + k * zrows,
                                               zrows)])

        @pl.when(s == 0)
        def _():
            pltpu.sync_copy(zbuf.at[pl.ds(0, 32)],
                            acc.at[pl.ds(16 * STRIPE, 32)])
        plsc.subcore_barrier()

        for cc in range(2):
            q = 2 * p + cc

            @pl.when(c == cc)
            def _():
                def win(w, _):
                    g = w * 16 + s
                    r0 = g * NCH
                    cps = [
                        pltpu.async_copy(dst_hbm.at[pl.ds(r0, NCH)],
                                         dstv, lsem),
                        pltpu.async_copy(src_hbm.at[pl.ds(r0, NCH)],
                                         srcv, lsem),
                        pltpu.async_copy(val_hbm.at[pl.ds(r0, NCH)],
                                         valv, lsem),
                    ]
                    for cp in cps:
                        cp.wait()
                    gps = [
                        pltpu.async_copy(
                            tab_hbm.at[q].at[srcv.at[i]],
                            rows.at[pl.ds(i * CH, CH)], gsem)
                        for i in range(NCH)
                    ]
                    sps = []
                    for i in range(NCH):
                        gps[i].wait()

                        def scale(m, _):
                            vv = valv[i, pl.ds(m * 16, 16)]
                            for k in range(16):
                                e = i * CH + m * 16 + k
                                rows[e, :] = vv[k] * rows[e, :]
                            return 0
                        lax.fori_loop(0, CH // 16, scale, 0)
                        sps.append(pltpu.async_copy(
                            rows.at[pl.ds(i * CH, CH)],
                            acc.at[dstv.at[i]], ssem,
                            add=True))
                    for cp in sps:
                        cp.wait()
                    return 0
                lax.fori_loop(0, WIN_PER_TILE, win, 0)

        plsc.subcore_barrier()

        for cc in range(2):
            q = 2 * p + cc

            @pl.when(c == cc)
            def _():
                pltpu.sync_copy(
                    acc.at[pl.ds(s * STRIPE, STRIPE)],
                    out_hbm.at[q].at[pl.ds(s * STRIPE, STRIPE)])

                @pl.when(s == 0)
                def _():
                    pltpu.sync_copy(
                        acc.at[pl.ds(16 * STRIPE, 32)],
                        out_hbm.at[q].at[pl.ds(16 * STRIPE, 32)])

        plsc.subcore_barrier()


@functools.partial(
    pl.kernel,
    out_type=jax.ShapeDtypeStruct((NQ, N_U, QL), jnp.float32),
    mesh=_mesh,
    compiler_params=_sc_params,
    scratch_types=[
        pltpu.VMEM_SHARED((N_U, QL), jnp.float32),
        pltpu.VMEM((NCH, CH), jnp.int32),
        pltpu.VMEM((NCH, CH), jnp.int32),
        pltpu.VMEM((NCH, CH), jnp.float32),
        pltpu.VMEM((EW, QL), jnp.float32),
        pltpu.VMEM((142, QL), jnp.float32),
        pltpu.SemaphoreType.DMA,
        pltpu.SemaphoreType.DMA,
        pltpu.SemaphoreType.DMA,
    ],
)
def _spmm(dst_hbm, src_hbm, val_hbm, tab_hbm, out_hbm,
          acc, dstv, srcv, valv, rows, zbuf, lsem, gsem, ssem):
    _spmm_body(dst_hbm, src_hbm, val_hbm, tab_hbm, out_hbm,
               acc, dstv, srcv, valv, rows, zbuf, lsem, gsem, ssem)


@functools.partial(
    pl.kernel,
    out_type=[jax.ShapeDtypeStruct((32, 128, QL), jnp.float32),
              jax.ShapeDtypeStruct((32, 128, QL), jnp.float32),
              jax.ShapeDtypeStruct((32, 128, QL), jnp.float32)],
    mesh=_mesh,
    compiler_params=_sc_params,
    scratch_types=(
        [pltpu.VMEM((1, 128), jnp.int32) for _ in range(3)]
        + [pltpu.VMEM((NQ, 128, QL), jnp.float32) for _ in range(9)]
        + [pltpu.VMEM((128, QL), jnp.float32) for _ in range(3)]
    ),
)
def _batch(uid_hbm, pos_hbm, neg_hbm,
           eu0, zu1, zu2, ei0, zi1, zi2,
           ps_hbm, ns_hbm, rg_hbm,
           iu, ip, inn,
           bu0, bu1, bu2, bp0, bp1, bp2, bn0, bn1, bn2,
           ops, ons, org):
    c = lax.axis_index("c")
    s = lax.axis_index("s")
    wid = 2 * s + c

    pltpu.sync_copy(uid_hbm.at[wid], iu)
    pltpu.sync_copy(pos_hbm.at[wid], ip)
    pltpu.sync_copy(neg_hbm.at[wid], inn)

    for q in range(NQ):
        pltpu.sync_copy(eu0.at[q].at[iu.at[0]], bu0.at[q])
        pltpu.sync_copy(zu1.at[q].at[iu.at[0]], bu1.at[q])
        pltpu.sync_copy(zu2.at[q].at[iu.at[0]], bu2.at[q])
        pltpu.sync_copy(ei0.at[q].at[ip.at[0]], bp0.at[q])
        pltpu.sync_copy(zi1.at[q].at[ip.at[0]], bp1.at[q])
        pltpu.sync_copy(zi2.at[q].at[ip.at[0]], bp2.at[q])
        pltpu.sync_copy(ei0.at[q].at[inn.at[0]], bn0.at[q])
        pltpu.sync_copy(zi1.at[q].at[inn.at[0]], bn1.at[q])
        pltpu.sync_copy(zi2.at[q].at[inn.at[0]], bn2.at[q])

    def sample(b, _):
        accp = jnp.zeros((QL,), jnp.float32)
        accn = jnp.zeros((QL,), jnp.float32)
        accr = jnp.zeros((QL,), jnp.float32)
        for q in range(NQ):
            u0 = bu0[q, b, :]
            p0 = bp0[q, b, :]
            n0 = bn0[q, b, :]
            u = u0 + bu1[q, b, :] + bu2[q, b, :]
            pv = p0 + bp1[q, b, :] + bp2[q, b, :]
            nv = n0 + bn1[q, b, :] + bn2[q, b, :]
            accp = accp + u * pv
            accn = accn + u * nv
            accr = accr + u0 * u0 + p0 * p0 + n0 * n0
        ops[b, :] = accp
        ons[b, :] = accn
        org[b, :] = accr
        return 0
    lax.fori_loop(0, 128, sample, 0)

    pltpu.sync_copy(ops, ps_hbm.at[wid])
    pltpu.sync_copy(ons, ns_hbm.at[wid])
    pltpu.sync_copy(org, rg_hbm.at[wid])


def _finalize_body(ps_ref, ns_ref, rg_ref, loss_ref, lossr_ref):
    d = jnp.sum(ps_ref[...] - ns_ref[...], axis=-1)
    loss_r = -jnp.mean(jnp.log(jax.nn.sigmoid(d)))
    loss_reg = 0.5 * jnp.sum(rg_ref[...]) / float(BATCH) * LAMBDA_1
    lossr_ref[0, 0] = loss_r
    loss_ref[0, 0] = loss_r + loss_reg


def kernel(uids, pos, neg, E_u_0, E_i_0, adj_row, adj_col, adj_val):
    padn = NNZ_P - NNZ
    pad_idx = (jnp.arange(padn, dtype=jnp.int32) * 61) % 99991
    rowp = jnp.concatenate([adj_row, pad_idx]).reshape(NNZ_P // CH, CH)
    colp = jnp.concatenate([adj_col, pad_idx]).reshape(NNZ_P // CH, CH)
    valp = jnp.concatenate([adj_val, jnp.zeros((padn,), jnp.float32)]
                           ).reshape(NNZ_P // CH, CH)

    eu0 = E_u_0.reshape(N_U, NQ, QL).transpose(1, 0, 2)
    ei0 = E_i_0.reshape(N_I, NQ, QL).transpose(1, 0, 2)

    zu1 = _spmm(rowp, colp, valp, ei0)
    zi1 = _spmm(colp, rowp, valp, eu0)
    zu2 = _spmm(rowp, colp, valp, zi1)
    zi2 = _spmm(colp, rowp, valp, zu1)

    u32 = uids.reshape(32, 1, 128)
    p32 = pos.reshape(32, 1, 128)
    n32 = neg.reshape(32, 1, 128)
    ps, ns, rg = _batch(u32, p32, n32, eu0, zu1, zu2, ei0, zi1, zi2)

    loss, loss_r = pl.pallas_call(
        _finalize_body,
        out_shape=[jax.ShapeDtypeStruct((1, 1), jnp.float32),
                   jax.ShapeDtypeStruct((1, 1), jnp.float32)],
        out_specs=[pl.BlockSpec(memory_space=pltpu.SMEM),
                   pl.BlockSpec(memory_space=pltpu.SMEM)],
    )(ps.reshape(BATCH, QL), ns.reshape(BATCH, QL), rg.reshape(BATCH, QL))
    return (loss[0, 0], loss_r[0, 0])


# speedup vs baseline: 9.8098x; 1.1983x over previous
import functools
import jax
import jax.numpy as jnp
from jax import lax
from jax.experimental import pallas as pl
from jax.experimental.pallas import tpu as pltpu
from jax.experimental.pallas import tpu_sc as plsc

N_U = 100000
N_I = 100000
D = 64
NQ = 4
QL = 16
NNZ = 1000000
NNZ_P = 1 << 20
EW = 512
NCH = 2
CH = EW // NCH
N_WIN = NNZ_P // EW
WIN_PER_TILE = N_WIN // 16
GRP = 8
N_GRP = WIN_PER_TILE // GRP
BATCH = 4096
LAMBDA_1 = 1e-4

_mesh = plsc.VectorSubcoreMesh(core_axis_name="c", subcore_axis_name="s")
_sc_params = pltpu.CompilerParams(use_tc_tiling_on_sc=False)


def _spmm_body(dst_hbm, src_hbm, val_hbm, tab_hbm, out_hbm,
               acc, dstv, srcv, valv, rows, zbuf, lsem, gsem, ssem):
    c = lax.axis_index("c")
    s = lax.axis_index("s")

    def _z(i, _):
        zbuf[i, :] = jnp.zeros((QL,), jnp.float32)
        return 0
    lax.fori_loop(0, zbuf.shape[0], _z, 0)

    zrows = zbuf.shape[0]
    STRIPE = 6248
    n_zcopy = STRIPE // zrows

    for p in range(2):
        def _zc(k, _):
            pltpu.sync_copy(zbuf, acc.at[pl.ds(s * STRIPE + k * zrows,
                                               zrows)])
            return 0
        lax.fori_loop(0, n_zcopy, _zc, 0)
        pltpu.sync_copy(zbuf.at[pl.ds(0, 8)],
                        acc.at[pl.ds(s * STRIPE + n_zcopy * zrows, 8)])

        @pl.when(s == 0)
        def _():
            pltpu.sync_copy(zbuf.at[pl.ds(0, 32)],
                            acc.at[pl.ds(16 * STRIPE, 32)])
        plsc.subcore_barrier()

        for cc in range(2):
            q = 2 * p + cc

            @pl.when(c == cc)
            def _():
                def issue_loads(t, j):
                    r0 = ((t * GRP + j) * 16 + s) * NCH
                    return [
                        pltpu.async_copy(dst_hbm.at[pl.ds(r0, NCH)],
                                         dstv.at[j], lsem),
                        pltpu.async_copy(src_hbm.at[pl.ds(r0, NCH)],
                                         srcv.at[j], lsem),
                        pltpu.async_copy(val_hbm.at[pl.ds(r0, NCH)],
                                         valv.at[j], lsem),
                    ]

                def issue_gathers(j, b):
                    return [
                        pltpu.async_copy(
                            tab_hbm.at[q].at[srcv.at[j].at[i]],
                            rows.at[b].at[pl.ds(i * CH, CH)], gsem)
                        for i in range(NCH)
                    ]

                def group(t, _):
                    lps = [issue_loads(t, j) for j in range(GRP)]
                    for cp in lps[0]:
                        cp.wait()
                    gps = issue_gathers(0, 0)
                    sps_prev = None
                    for j in range(GRP):
                        b = j % 2
                        nxt = None
                        if j < GRP - 1:
                            if sps_prev is not None:
                                for cp in sps_prev:
                                    cp.wait()
                                sps_prev = None
                            for cp in lps[j + 1]:
                                cp.wait()
                            nxt = issue_gathers(j + 1, 1 - b)
                        sps = []
                        for i in range(NCH):
                            gps[i].wait()

                            def scale(m, _):
                                vv = valv[j, i, pl.ds(m * 16, 16)]
                                for k in range(16):
                                    e = i * CH + m * 16 + k
                                    rows[b, e, :] = vv[k] * rows[b, e, :]
                                return 0
                            lax.fori_loop(0, CH // 16, scale, 0)
                            sps.append(pltpu.async_copy(
                                rows.at[b].at[pl.ds(i * CH, CH)],
                                acc.at[dstv.at[j].at[i]], ssem,
                                add=True))
                        if sps_prev is not None:
                            for cp in sps_prev:
                                cp.wait()
                        sps_prev = sps
                        gps = nxt
                    for cp in sps_prev:
                        cp.wait()
                    return 0
                lax.fori_loop(0, N_GRP, group, 0)

        plsc.subcore_barrier()

        for cc in range(2):
            q = 2 * p + cc

            @pl.when(c == cc)
            def _():
                pltpu.sync_copy(
                    acc.at[pl.ds(s * STRIPE, STRIPE)],
                    out_hbm.at[q].at[pl.ds(s * STRIPE, STRIPE)])

                @pl.when(s == 0)
                def _():
                    pltpu.sync_copy(
                        acc.at[pl.ds(16 * STRIPE, 32)],
                        out_hbm.at[q].at[pl.ds(16 * STRIPE, 32)])

        plsc.subcore_barrier()


@functools.partial(
    pl.kernel,
    out_type=jax.ShapeDtypeStruct((NQ, N_U, QL), jnp.float32),
    mesh=_mesh,
    compiler_params=_sc_params,
    scratch_types=[
        pltpu.VMEM_SHARED((N_U, QL), jnp.float32),
        pltpu.VMEM((GRP, NCH, CH), jnp.int32),
        pltpu.VMEM((GRP, NCH, CH), jnp.int32),
        pltpu.VMEM((GRP, NCH, CH), jnp.float32),
        pltpu.VMEM((2, EW, QL), jnp.float32),
        pltpu.VMEM((40, QL), jnp.float32),
        pltpu.SemaphoreType.DMA,
        pltpu.SemaphoreType.DMA,
        pltpu.SemaphoreType.DMA,
    ],
)
def _spmm(dst_hbm, src_hbm, val_hbm, tab_hbm, out_hbm,
          acc, dstv, srcv, valv, rows, zbuf, lsem, gsem, ssem):
    _spmm_body(dst_hbm, src_hbm, val_hbm, tab_hbm, out_hbm,
               acc, dstv, srcv, valv, rows, zbuf, lsem, gsem, ssem)


@functools.partial(
    pl.kernel,
    out_type=[jax.ShapeDtypeStruct((32, 128, QL), jnp.float32),
              jax.ShapeDtypeStruct((32, 128, QL), jnp.float32),
              jax.ShapeDtypeStruct((32, 128, QL), jnp.float32)],
    mesh=_mesh,
    compiler_params=_sc_params,
    scratch_types=(
        [pltpu.VMEM((1, 128), jnp.int32) for _ in range(3)]
        + [pltpu.VMEM((NQ, 128, QL), jnp.float32) for _ in range(9)]
        + [pltpu.VMEM((128, QL), jnp.float32) for _ in range(3)]
    ),
)
def _batch(uid_hbm, pos_hbm, neg_hbm,
           eu0, zu1, zu2, ei0, zi1, zi2,
           ps_hbm, ns_hbm, rg_hbm,
           iu, ip, inn,
           bu0, bu1, bu2, bp0, bp1, bp2, bn0, bn1, bn2,
           ops, ons, org):
    c = lax.axis_index("c")
    s = lax.axis_index("s")
    wid = 2 * s + c

    pltpu.sync_copy(uid_hbm.at[wid], iu)
    pltpu.sync_copy(pos_hbm.at[wid], ip)
    pltpu.sync_copy(neg_hbm.at[wid], inn)

    for q in range(NQ):
        pltpu.sync_copy(eu0.at[q].at[iu.at[0]], bu0.at[q])
        pltpu.sync_copy(zu1.at[q].at[iu.at[0]], bu1.at[q])
        pltpu.sync_copy(zu2.at[q].at[iu.at[0]], bu2.at[q])
        pltpu.sync_copy(ei0.at[q].at[ip.at[0]], bp0.at[q])
        pltpu.sync_copy(zi1.at[q].at[ip.at[0]], bp1.at[q])
        pltpu.sync_copy(zi2.at[q].at[ip.at[0]], bp2.at[q])
        pltpu.sync_copy(ei0.at[q].at[inn.at[0]], bn0.at[q])
        pltpu.sync_copy(zi1.at[q].at[inn.at[0]], bn1.at[q])
        pltpu.sync_copy(zi2.at[q].at[inn.at[0]], bn2.at[q])

    def sample(b, _):
        accp = jnp.zeros((QL,), jnp.float32)
        accn = jnp.zeros((QL,), jnp.float32)
        accr = jnp.zeros((QL,), jnp.float32)
        for q in range(NQ):
            u0 = bu0[q, b, :]
            p0 = bp0[q, b, :]
            n0 = bn0[q, b, :]
            u = u0 + bu1[q, b, :] + bu2[q, b, :]
            pv = p0 + bp1[q, b, :] + bp2[q, b, :]
            nv = n0 + bn1[q, b, :] + bn2[q, b, :]
            accp = accp + u * pv
            accn = accn + u * nv
            accr = accr + u0 * u0 + p0 * p0 + n0 * n0
        ops[b, :] = accp
        ons[b, :] = accn
        org[b, :] = accr
        return 0
    lax.fori_loop(0, 128, sample, 0)

    pltpu.sync_copy(ops, ps_hbm.at[wid])
    pltpu.sync_copy(ons, ns_hbm.at[wid])
    pltpu.sync_copy(org, rg_hbm.at[wid])


def _finalize_body(ps_ref, ns_ref, rg_ref, loss_ref, lossr_ref):
    d = jnp.sum(ps_ref[...] - ns_ref[...], axis=-1)
    loss_r = -jnp.mean(jnp.log(jax.nn.sigmoid(d)))
    loss_reg = 0.5 * jnp.sum(rg_ref[...]) / float(BATCH) * LAMBDA_1
    lossr_ref[0, 0] = loss_r
    loss_ref[0, 0] = loss_r + loss_reg


def kernel(uids, pos, neg, E_u_0, E_i_0, adj_row, adj_col, adj_val):
    padn = NNZ_P - NNZ
    pad_idx = (jnp.arange(padn, dtype=jnp.int32) * 61) % 99991
    rowp = jnp.concatenate([adj_row, pad_idx]).reshape(NNZ_P // CH, CH)
    colp = jnp.concatenate([adj_col, pad_idx]).reshape(NNZ_P // CH, CH)
    valp = jnp.concatenate([adj_val, jnp.zeros((padn,), jnp.float32)]
                           ).reshape(NNZ_P // CH, CH)

    eu0 = E_u_0.reshape(N_U, NQ, QL).transpose(1, 0, 2)
    ei0 = E_i_0.reshape(N_I, NQ, QL).transpose(1, 0, 2)

    zu1 = _spmm(rowp, colp, valp, ei0)
    zi1 = _spmm(colp, rowp, valp, eu0)
    zu2 = _spmm(rowp, colp, valp, zi1)
    zi2 = _spmm(colp, rowp, valp, zu1)

    u32 = uids.reshape(32, 1, 128)
    p32 = pos.reshape(32, 1, 128)
    n32 = neg.reshape(32, 1, 128)
    ps, ns, rg = _batch(u32, p32, n32, eu0, zu1, zu2, ei0, zi1, zi2)

    loss, loss_r = pl.pallas_call(
        _finalize_body,
        out_shape=[jax.ShapeDtypeStruct((1, 1), jnp.float32),
                   jax.ShapeDtypeStruct((1, 1), jnp.float32)],
        out_specs=[pl.BlockSpec(memory_space=pltpu.SMEM),
                   pl.BlockSpec(memory_space=pltpu.SMEM)],
    )(ps.reshape(BATCH, QL), ns.reshape(BATCH, QL), rg.reshape(BATCH, QL))
    return (loss[0, 0], loss_r[0, 0])


# speedup vs baseline: 10.1547x; 1.0352x over previous
import functools
import jax
import jax.numpy as jnp
from jax import lax
from jax.experimental import pallas as pl
from jax.experimental.pallas import tpu as pltpu
from jax.experimental.pallas import tpu_sc as plsc

N_U = 100000
N_I = 100000
D = 64
NQ = 4
QL = 16
NNZ = 1000000
NNZ_P = 1 << 20
EW = 512
NCH = 2
CH = EW // NCH
N_WIN = NNZ_P // EW
WIN_PER_TILE = N_WIN // 16
GRP = 8
N_GRP = WIN_PER_TILE // GRP
BATCH = 4096
LAMBDA_1 = 1e-4

_mesh = plsc.VectorSubcoreMesh(core_axis_name="c", subcore_axis_name="s")
_sc_params = pltpu.CompilerParams(use_tc_tiling_on_sc=False)


def _spmm_body(dst_hbm, src_hbm, val_hbm, tab_hbm, out_hbm,
               acc, dstv, srcv, valv, rows, zbuf, lsem, gsem, ssem):
    c = lax.axis_index("c")
    s = lax.axis_index("s")

    def _z(i, _):
        zbuf[i, :] = jnp.zeros((QL,), jnp.float32)
        return 0
    lax.fori_loop(0, zbuf.shape[0], _z, 0)

    zrows = zbuf.shape[0]
    STRIPE = 6248
    n_zcopy = STRIPE // zrows

    for p in range(2):
        def _zc(k, _):
            pltpu.sync_copy(zbuf, acc.at[pl.ds(s * STRIPE + k * zrows,
                                               zrows)])
            return 0
        lax.fori_loop(0, n_zcopy, _zc, 0)
        pltpu.sync_copy(zbuf.at[pl.ds(0, 8)],
                        acc.at[pl.ds(s * STRIPE + n_zcopy * zrows, 8)])

        @pl.when(s == 0)
        def _():
            pltpu.sync_copy(zbuf.at[pl.ds(0, 32)],
                            acc.at[pl.ds(16 * STRIPE, 32)])
        plsc.subcore_barrier()

        for cc in range(2):
            q = 2 * p + cc

            @pl.when(c == cc)
            def _():
                def issue_loads(t, j):
                    r0 = ((t * GRP + j) * 16 + s) * NCH
                    return [
                        pltpu.async_copy(dst_hbm.at[pl.ds(r0, NCH)],
                                         dstv.at[j], lsem),
                        pltpu.async_copy(src_hbm.at[pl.ds(r0, NCH)],
                                         srcv.at[j], lsem),
                        pltpu.async_copy(val_hbm.at[pl.ds(r0, NCH)],
                                         valv.at[j], lsem),
                    ]

                def issue_gathers(j, b):
                    return [
                        pltpu.async_copy(
                            tab_hbm.at[q].at[srcv.at[j].at[i]],
                            rows.at[b].at[pl.ds(i * CH, CH)], gsem)
                        for i in range(NCH)
                    ]

                def group(t, _):
                    lps = [issue_loads(t, j) for j in range(GRP)]
                    for cp in lps[0]:
                        cp.wait()
                    gps = issue_gathers(0, 0)
                    sps_prev = None
                    for j in range(GRP):
                        b = j % 2
                        nxt = None
                        if j < GRP - 1:
                            if sps_prev is not None:
                                for cp in sps_prev:
                                    cp.wait()
                                sps_prev = None
                            for cp in lps[j + 1]:
                                cp.wait()
                            nxt = issue_gathers(j + 1, 1 - b)
                        sps = []
                        for i in range(NCH):
                            gps[i].wait()

                            @plsc.parallel_loop(0, CH // 16, 1, unroll=2)
                            def scale(m, _j=j, _i=i, _b=b):
                                vv = valv[_j, _i, pl.ds(m * 16, 16)]
                                for k in range(16):
                                    e = _i * CH + m * 16 + k
                                    rows[_b, e, :] = vv[k] * rows[_b, e, :]
                            sps.append(pltpu.async_copy(
                                rows.at[b].at[pl.ds(i * CH, CH)],
                                acc.at[dstv.at[j].at[i]], ssem,
                                add=True))
                        if sps_prev is not None:
                            for cp in sps_prev:
                                cp.wait()
                        sps_prev = sps
                        gps = nxt
                    for cp in sps_prev:
                        cp.wait()
                    return 0
                lax.fori_loop(0, N_GRP, group, 0)

        plsc.subcore_barrier()

        for cc in range(2):
            q = 2 * p + cc

            @pl.when(c == cc)
            def _():
                pltpu.sync_copy(
                    acc.at[pl.ds(s * STRIPE, STRIPE)],
                    out_hbm.at[q].at[pl.ds(s * STRIPE, STRIPE)])

                @pl.when(s == 0)
                def _():
                    pltpu.sync_copy(
                        acc.at[pl.ds(16 * STRIPE, 32)],
                        out_hbm.at[q].at[pl.ds(16 * STRIPE, 32)])

        plsc.subcore_barrier()


@functools.partial(
    pl.kernel,
    out_type=jax.ShapeDtypeStruct((NQ, N_U, QL), jnp.float32),
    mesh=_mesh,
    compiler_params=_sc_params,
    scratch_types=[
        pltpu.VMEM_SHARED((N_U, QL), jnp.float32),
        pltpu.VMEM((GRP, NCH, CH), jnp.int32),
        pltpu.VMEM((GRP, NCH, CH), jnp.int32),
        pltpu.VMEM((GRP, NCH, CH), jnp.float32),
        pltpu.VMEM((2, EW, QL), jnp.float32),
        pltpu.VMEM((40, QL), jnp.float32),
        pltpu.SemaphoreType.DMA,
        pltpu.SemaphoreType.DMA,
        pltpu.SemaphoreType.DMA,
    ],
)
def _spmm(dst_hbm, src_hbm, val_hbm, tab_hbm, out_hbm,
          acc, dstv, srcv, valv, rows, zbuf, lsem, gsem, ssem):
    _spmm_body(dst_hbm, src_hbm, val_hbm, tab_hbm, out_hbm,
               acc, dstv, srcv, valv, rows, zbuf, lsem, gsem, ssem)


@functools.partial(
    pl.kernel,
    out_type=[jax.ShapeDtypeStruct((32, 128, QL), jnp.float32),
              jax.ShapeDtypeStruct((32, 128, QL), jnp.float32),
              jax.ShapeDtypeStruct((32, 128, QL), jnp.float32)],
    mesh=_mesh,
    compiler_params=_sc_params,
    scratch_types=(
        [pltpu.VMEM((1, 128), jnp.int32) for _ in range(3)]
        + [pltpu.VMEM((NQ, 128, QL), jnp.float32) for _ in range(9)]
        + [pltpu.VMEM((128, QL), jnp.float32) for _ in range(3)]
    ),
)
def _batch(uid_hbm, pos_hbm, neg_hbm,
           eu0, zu1, zu2, ei0, zi1, zi2,
           ps_hbm, ns_hbm, rg_hbm,
           iu, ip, inn,
           bu0, bu1, bu2, bp0, bp1, bp2, bn0, bn1, bn2,
           ops, ons, org):
    c = lax.axis_index("c")
    s = lax.axis_index("s")
    wid = 2 * s + c

    pltpu.sync_copy(uid_hbm.at[wid], iu)
    pltpu.sync_copy(pos_hbm.at[wid], ip)
    pltpu.sync_copy(neg_hbm.at[wid], inn)

    for q in range(NQ):
        pltpu.sync_copy(eu0.at[q].at[iu.at[0]], bu0.at[q])
        pltpu.sync_copy(zu1.at[q].at[iu.at[0]], bu1.at[q])
        pltpu.sync_copy(zu2.at[q].at[iu.at[0]], bu2.at[q])
        pltpu.sync_copy(ei0.at[q].at[ip.at[0]], bp0.at[q])
        pltpu.sync_copy(zi1.at[q].at[ip.at[0]], bp1.at[q])
        pltpu.sync_copy(zi2.at[q].at[ip.at[0]], bp2.at[q])
        pltpu.sync_copy(ei0.at[q].at[inn.at[0]], bn0.at[q])
        pltpu.sync_copy(zi1.at[q].at[inn.at[0]], bn1.at[q])
        pltpu.sync_copy(zi2.at[q].at[inn.at[0]], bn2.at[q])

    def sample(b, _):
        accp = jnp.zeros((QL,), jnp.float32)
        accn = jnp.zeros((QL,), jnp.float32)
        accr = jnp.zeros((QL,), jnp.float32)
        for q in range(NQ):
            u0 = bu0[q, b, :]
            p0 = bp0[q, b, :]
            n0 = bn0[q, b, :]
            u = u0 + bu1[q, b, :] + bu2[q, b, :]
            pv = p0 + bp1[q, b, :] + bp2[q, b, :]
            nv = n0 + bn1[q, b, :] + bn2[q, b, :]
            accp = accp + u * pv
            accn = accn + u * nv
            accr = accr + u0 * u0 + p0 * p0 + n0 * n0
        ops[b, :] = accp
        ons[b, :] = accn
        org[b, :] = accr
        return 0
    lax.fori_loop(0, 128, sample, 0)

    pltpu.sync_copy(ops, ps_hbm.at[wid])
    pltpu.sync_copy(ons, ns_hbm.at[wid])
    pltpu.sync_copy(org, rg_hbm.at[wid])


def _finalize_body(ps_ref, ns_ref, rg_ref, loss_ref, lossr_ref):
    d = jnp.sum(ps_ref[...] - ns_ref[...], axis=-1)
    loss_r = -jnp.mean(jnp.log(jax.nn.sigmoid(d)))
    loss_reg = 0.5 * jnp.sum(rg_ref[...]) / float(BATCH) * LAMBDA_1
    lossr_ref[0, 0] = loss_r
    loss_ref[0, 0] = loss_r + loss_reg


def kernel(uids, pos, neg, E_u_0, E_i_0, adj_row, adj_col, adj_val):
    padn = NNZ_P - NNZ
    pad_idx = (jnp.arange(padn, dtype=jnp.int32) * 61) % 99991
    rowp = jnp.concatenate([adj_row, pad_idx]).reshape(NNZ_P // CH, CH)
    colp = jnp.concatenate([adj_col, pad_idx]).reshape(NNZ_P // CH, CH)
    valp = jnp.concatenate([adj_val, jnp.zeros((padn,), jnp.float32)]
                           ).reshape(NNZ_P // CH, CH)

    eu0 = E_u_0.reshape(N_U, NQ, QL).transpose(1, 0, 2)
    ei0 = E_i_0.reshape(N_I, NQ, QL).transpose(1, 0, 2)

    zu1 = _spmm(rowp, colp, valp, ei0)
    zi1 = _spmm(colp, rowp, valp, eu0)
    zu2 = _spmm(rowp, colp, valp, zi1)
    zi2 = _spmm(colp, rowp, valp, zu1)

    u32 = uids.reshape(32, 1, 128)
    p32 = pos.reshape(32, 1, 128)
    n32 = neg.reshape(32, 1, 128)
    ps, ns, rg = _batch(u32, p32, n32, eu0, zu1, zu2, ei0, zi1, zi2)

    loss, loss_r = pl.pallas_call(
        _finalize_body,
        out_shape=[jax.ShapeDtypeStruct((1, 1), jnp.float32),
                   jax.ShapeDtypeStruct((1, 1), jnp.float32)],
        out_specs=[pl.BlockSpec(memory_space=pltpu.SMEM),
                   pl.BlockSpec(memory_space=pltpu.SMEM)],
    )(ps.reshape(BATCH, QL), ns.reshape(BATCH, QL), rg.reshape(BATCH, QL))
    return (loss[0, 0], loss_r[0, 0])
